# baseline (speedup 1.0000x reference)
import jax
import jax.numpy as jnp
from jax.experimental import pallas as pl
from jax.experimental.pallas import tpu as pltpu

_N = 5000
_PAD = 5120
_CH = 8
_NCHUNK = _N // _CH
_BIGIDX = 3e7


def _loss_kernel(pxyzT_ref, pprob_ref, txyz_ref, praw_ref, traw_ref,
                 tprob_ref, out_ref):
    px = pxyzT_ref[0:1, :]
    py = pxyzT_ref[1:2, :]
    pz = pxyzT_ref[2:3, :]
    ppr = pprob_ref[0:1, :]
    b2 = px * px + py * py + pz * pz
    b2 = jnp.where(ppr >= 0.5, b2, jnp.float32(1e9))
    b2blk0 = jnp.broadcast_to(b2, (_CH, _PAD))
    lanef = jax.lax.broadcasted_iota(jnp.int32, (_CH, _PAD), 1).astype(jnp.float32)
    row_i = jax.lax.broadcasted_iota(jnp.int32, (_CH, 1), 0)
    nt = jnp.sum(jnp.where(tprob_ref[0] >= 0.5, jnp.float32(1.0), jnp.float32(0.0)))

    def argmin_rows(r0):
        mv = jnp.min(r0, axis=1, keepdims=True)
        mb = jnp.broadcast_to(mv, (_CH, _PAD))
        idxf = jnp.where(r0 == mb, lanef, _BIGIDX)
        iv = jnp.min(idxf, axis=1, keepdims=True)
        return mv, iv

    def dist_block(c, b2blk):
        base = c * _CH
        t8 = txyz_ref[pl.ds(base, _CH), :]
        dot = jax.lax.dot_general(
            t8, pxyzT_ref[...],
            dimension_numbers=(((1,), (0,)), ((), ())),
            preferred_element_type=jnp.float32)
        tbase = base * 4
        a2v = jnp.zeros((_CH, 1), dtype=jnp.float32)
        a2s = []
        for k in range(_CH):
            off = tbase + 4 * k
            tx = traw_ref[off]
            ty = traw_ref[off + 1]
            tz = traw_ref[off + 2]
            tp = traw_ref[off + 3]
            a2k = tx * tx + ty * ty + tz * tz
            a2s.append(a2k)
            a2k = a2k + jnp.where(tp >= 0.5, jnp.float32(0.0), jnp.inf)
            a2v = jnp.where(row_i == k, a2k, a2v)
        d2 = (a2v + b2blk) - 2.0 * dot
        dist = jnp.sqrt(jnp.maximum(d2, 0.0))
        return jnp.where(dist <= 1.0, dist, jnp.inf)

    def accum(k, base4, b_k, matched, nm, sdn, smse):
        off = base4 + 4 * k
        tx = traw_ref[off]
        ty = traw_ref[off + 1]
        tz = traw_ref[off + 2]
        tp = traw_ref[off + 3]
        idxi = jnp.maximum(b_k, jnp.float32(0.0)).astype(jnp.int32)
        pbase = idxi * 4
        pxi = praw_ref[pbase]
        pyi = praw_ref[pbase + 1]
        pzi = praw_ref[pbase + 2]
        ppi = praw_ref[pbase + 3]
        dx = tx - pxi
        dy = ty - pyi
        dz = tz - pzi
        dn = jnp.sqrt(dx * dx + dy * dy + dz * dz)
        mf = jnp.where(matched, jnp.float32(1.0), jnp.float32(0.0))
        dp = tp - ppi
        return nm + mf, sdn + mf * dn, smse + mf * dp * dp

    def chunk(c, carry):
        b2blk0c, nm0, sdn0, smse0 = carry
        base4 = c * (_CH * 4)
        r0 = dist_block(c, b2blk0c)
        mv8, iv8 = argmin_rows(r0)
        spec = [(mv8[k, 0], iv8[k, 0]) for k in range(_CH)]

        mats = [mv < jnp.inf for mv, _ in spec]
        bsp = [jnp.where(mats[k], spec[k][1], jnp.float32(-1.0))
               for k in range(_CH)]
        coll = jnp.bool_(False)
        for k in range(1, _CH):
            ck = jnp.bool_(False)
            for j in range(k):
                ck = jnp.logical_or(ck, spec[k][1] == bsp[j])
            coll = jnp.logical_or(coll, jnp.logical_and(mats[k], ck))

        def fast(op):
            b2blk, nm, sdn, smse = op
            masks = [lanef == bsp[k] for k in range(_CH)]
            while len(masks) > 1:
                nxt = [masks[i] | masks[i + 1] for i in range(0, len(masks) - 1, 2)]
                if len(masks) % 2:
                    nxt.append(masks[-1])
                masks = nxt
            b2blk = jnp.where(masks[0], jnp.float32(1e9), b2blk)
            for k in range(_CH):
                nm, sdn, smse = accum(k, base4, bsp[k], mats[k], nm, sdn, smse)
            return b2blk, nm, sdn, smse

        def slow(op):
            b2blk, nm, sdn, smse = op
            bs = []
            for k in range(_CH):
                mv, ix = spec[k]
                ck = jnp.bool_(False)
                for j in range(k):
                    ck = jnp.logical_or(ck, ix == bs[j])
                ck = jnp.logical_and(mats[k], ck)

                def redo(bb, kk=k):
                    rr = dist_block(c, bb)
                    mv2, iv2 = argmin_rows(rr)
                    return mv2[kk, 0], iv2[kk, 0]

                def keep(bb, mv=mv, ix=ix):
                    return mv, ix

                mv, ix = jax.lax.cond(ck, redo, keep, b2blk)
                matched = mv < jnp.inf
                b_k = jnp.where(matched, ix, jnp.float32(-1.0))
                bs.append(b_k)
                b2blk = jnp.where(lanef == b_k, jnp.float32(1e9), b2blk)
                nm, sdn, smse = accum(k, base4, b_k, matched, nm, sdn, smse)
            return b2blk, nm, sdn, smse

        return jax.lax.cond(coll, slow, fast, (b2blk0c, nm0, sdn0, smse0))

    z = jnp.float32(0.0)
    _, nm, sdn, smse = jax.lax.fori_loop(
        0, _NCHUNK, chunk, (b2blk0, z, z, z))
    nu = nt - nm
    denom = jnp.maximum(nm, 1.0)
    extra = jnp.where(nm > 0.0, sdn / denom + smse / denom, 0.0)
    out_ref[0, 0] = 10.0 * nu + nu + extra


def kernel(pred_cloud, true_cloud):
    zpad = jnp.zeros((_PAD - _N, 4), dtype=pred_cloud.dtype)
    ppad = jnp.concatenate([pred_cloud, zpad], axis=0)
    pxyzT = jnp.transpose(ppad[:, :3])
    pprob = ppad[:, 3].reshape(1, _PAD)
    txyz = true_cloud[:, :3]
    tprob = jnp.concatenate(
        [true_cloud[:, 3], jnp.full((_PAD - _N,), -1.0, dtype=true_cloud.dtype)]
    ).reshape(1, 40, 128)
    out = pl.pallas_call(
        _loss_kernel,
        out_shape=jax.ShapeDtypeStruct((1, 1), jnp.float32),
        in_specs=[
            pl.BlockSpec(memory_space=pltpu.VMEM),
            pl.BlockSpec(memory_space=pltpu.VMEM),
            pl.BlockSpec(memory_space=pltpu.VMEM),
            pl.BlockSpec(memory_space=pltpu.SMEM),
            pl.BlockSpec(memory_space=pltpu.SMEM),
            pl.BlockSpec(memory_space=pltpu.VMEM),
        ],
        out_specs=pl.BlockSpec(memory_space=pltpu.SMEM),
    )(pxyzT, pprob, txyz, pred_cloud.reshape(-1), true_cloud.reshape(-1),
      tprob)
    return out.reshape(())


# speedup vs baseline: 1.2675x; 1.2675x over previous
import jax
import jax.numpy as jnp
from jax.experimental import pallas as pl
from jax.experimental.pallas import tpu as pltpu

_N = 5000
_PAD = 5120
_CH = 8
_NCHUNK = _N // _CH
_BIGIDX = 3e7


def _loss_kernel(pxyzT_ref, pprob_ref, txyz_ref, praw_ref, traw_ref,
                 tprob_ref, out_ref):
    px = pxyzT_ref[0:1, :]
    py = pxyzT_ref[1:2, :]
    pz = pxyzT_ref[2:3, :]
    ppr = pprob_ref[0:1, :]
    b2 = px * px + py * py + pz * pz
    b2 = jnp.where(ppr >= 0.5, b2, jnp.float32(1e9))
    b2blk0 = jnp.broadcast_to(b2, (_CH, _PAD))
    lanef = jax.lax.broadcasted_iota(jnp.int32, (_CH, _PAD), 1).astype(jnp.float32)
    row_i = jax.lax.broadcasted_iota(jnp.int32, (_CH, 1), 0)
    nt = jnp.sum(jnp.where(tprob_ref[0] >= 0.5, jnp.float32(1.0), jnp.float32(0.0)))

    def argmin_rows(r0):
        mv = jnp.min(r0, axis=1, keepdims=True)
        mb = jnp.broadcast_to(mv, (_CH, _PAD))
        idxf = jnp.where(r0 == mb, lanef, _BIGIDX)
        iv = jnp.min(idxf, axis=1, keepdims=True)
        return mv, iv

    def dist_block(c):
        base = c * _CH
        t8 = txyz_ref[pl.ds(base, _CH), :]
        dot = jax.lax.dot_general(
            t8, pxyzT_ref[...],
            dimension_numbers=(((1,), (0,)), ((), ())),
            preferred_element_type=jnp.float32)
        tbase = base * 4
        a2v = jnp.zeros((_CH, 1), dtype=jnp.float32)
        a2s = []
        for k in range(_CH):
            off = tbase + 4 * k
            tx = traw_ref[off]
            ty = traw_ref[off + 1]
            tz = traw_ref[off + 2]
            tp = traw_ref[off + 3]
            a2k = tx * tx + ty * ty + tz * tz
            a2s.append(a2k)
            a2k = a2k + jnp.where(tp >= 0.5, jnp.float32(0.0), jnp.inf)
            a2v = jnp.where(row_i == k, a2k, a2v)
        d2 = (a2v + b2blk0) - 2.0 * dot
        dist = jnp.sqrt(jnp.maximum(d2, 0.0))
        return jnp.where(dist <= 1.0, dist, jnp.inf)

    def accum(k, base4, b_k, matched, nm, sdn, smse):
        off = base4 + 4 * k
        tx = traw_ref[off]
        ty = traw_ref[off + 1]
        tz = traw_ref[off + 2]
        tp = traw_ref[off + 3]
        idxi = jnp.maximum(b_k, jnp.float32(0.0)).astype(jnp.int32)
        pbase = idxi * 4
        pxi = praw_ref[pbase]
        pyi = praw_ref[pbase + 1]
        pzi = praw_ref[pbase + 2]
        ppi = praw_ref[pbase + 3]
        dx = tx - pxi
        dy = ty - pyi
        dz = tz - pzi
        dn = jnp.sqrt(dx * dx + dy * dy + dz * dz)
        mf = jnp.where(matched, jnp.float32(1.0), jnp.float32(0.0))
        dp = tp - ppi
        return nm + mf, sdn + mf * dn, smse + mf * dp * dp

    def chunk(c, carry):
        r0, takenv0, nm0, sdn0, smse0 = carry
        base4 = c * (_CH * 4)
        r0next = dist_block(jnp.minimum(c + 1, _NCHUNK - 1))
        mv8, iv8 = argmin_rows(r0 + takenv0)
        spec = [(mv8[k, 0], iv8[k, 0]) for k in range(_CH)]

        mats = [mv < jnp.inf for mv, _ in spec]
        bsp = [jnp.where(mats[k], spec[k][1], jnp.float32(-1.0))
               for k in range(_CH)]
        coll = jnp.bool_(False)
        for k in range(1, _CH):
            ck = jnp.bool_(False)
            for j in range(k):
                ck = jnp.logical_or(ck, spec[k][1] == bsp[j])
            coll = jnp.logical_or(coll, jnp.logical_and(mats[k], ck))

        def fast(op):
            takenv, nm, sdn, smse = op
            masks = [lanef == bsp[k] for k in range(_CH)]
            while len(masks) > 1:
                nxt = [masks[i] | masks[i + 1] for i in range(0, len(masks) - 1, 2)]
                if len(masks) % 2:
                    nxt.append(masks[-1])
                masks = nxt
            takenv = jnp.where(masks[0], jnp.inf, takenv)
            for k in range(_CH):
                nm, sdn, smse = accum(k, base4, bsp[k], mats[k], nm, sdn, smse)
            return takenv, nm, sdn, smse

        def slow(op):
            takenv, nm, sdn, smse = op
            bs = []
            for k in range(_CH):
                mv, ix = spec[k]
                ck = jnp.bool_(False)
                for j in range(k):
                    ck = jnp.logical_or(ck, ix == bs[j])
                ck = jnp.logical_and(mats[k], ck)

                def redo(tv, kk=k):
                    mv2, iv2 = argmin_rows(r0 + tv)
                    return mv2[kk, 0], iv2[kk, 0]

                def keep(tv, mv=mv, ix=ix):
                    return mv, ix

                mv, ix = jax.lax.cond(ck, redo, keep, takenv)
                matched = mv < jnp.inf
                b_k = jnp.where(matched, ix, jnp.float32(-1.0))
                bs.append(b_k)
                takenv = jnp.where(lanef == b_k, jnp.inf, takenv)
                nm, sdn, smse = accum(k, base4, b_k, matched, nm, sdn, smse)
            return takenv, nm, sdn, smse

        takenv, nm, sdn, smse = jax.lax.cond(
            coll, slow, fast, (takenv0, nm0, sdn0, smse0))
        return r0next, takenv, nm, sdn, smse

    z = jnp.float32(0.0)
    takenv_init = jnp.zeros((_CH, _PAD), dtype=jnp.float32)
    _, _, nm, sdn, smse = jax.lax.fori_loop(
        0, _NCHUNK, chunk, (dist_block(0), takenv_init, z, z, z))
    nu = nt - nm
    denom = jnp.maximum(nm, 1.0)
    extra = jnp.where(nm > 0.0, sdn / denom + smse / denom, 0.0)
    out_ref[0, 0] = 10.0 * nu + nu + extra


def kernel(pred_cloud, true_cloud):
    zpad = jnp.zeros((_PAD - _N, 4), dtype=pred_cloud.dtype)
    ppad = jnp.concatenate([pred_cloud, zpad], axis=0)
    pxyzT = jnp.transpose(ppad[:, :3])
    pprob = ppad[:, 3].reshape(1, _PAD)
    txyz = true_cloud[:, :3]
    tprob = jnp.concatenate(
        [true_cloud[:, 3], jnp.full((_PAD - _N,), -1.0, dtype=true_cloud.dtype)]
    ).reshape(1, 40, 128)
    out = pl.pallas_call(
        _loss_kernel,
        out_shape=jax.ShapeDtypeStruct((1, 1), jnp.float32),
        in_specs=[
            pl.BlockSpec(memory_space=pltpu.VMEM),
            pl.BlockSpec(memory_space=pltpu.VMEM),
            pl.BlockSpec(memory_space=pltpu.VMEM),
            pl.BlockSpec(memory_space=pltpu.SMEM),
            pl.BlockSpec(memory_space=pltpu.SMEM),
            pl.BlockSpec(memory_space=pltpu.VMEM),
        ],
        out_specs=pl.BlockSpec(memory_space=pltpu.SMEM),
    )(pxyzT, pprob, txyz, pred_cloud.reshape(-1), true_cloud.reshape(-1),
      tprob)
    return out.reshape(())


# speedup vs baseline: 1.5413x; 1.2160x over previous
import jax
import jax.numpy as jnp
from jax.experimental import pallas as pl
from jax.experimental.pallas import tpu as pltpu

_N = 5000
_PAD = 5120
_CH = 8
_NCHUNK = _N // _CH
_BIGIDX = 3e7


def _loss_kernel(pxyzT_ref, pprob_ref, txyz_ref, praw_ref, traw_ref,
                 tprob_ref, out_ref):
    px = pxyzT_ref[0:1, :]
    py = pxyzT_ref[1:2, :]
    pz = pxyzT_ref[2:3, :]
    ppr = pprob_ref[0:1, :]
    b2 = px * px + py * py + pz * pz
    b2 = jnp.where(ppr >= 0.5, b2, jnp.float32(1e9))
    b2blk0 = jnp.broadcast_to(b2, (_CH, _PAD))
    lanef = jax.lax.broadcasted_iota(jnp.int32, (_CH, _PAD), 1).astype(jnp.float32)
    row_i = jax.lax.broadcasted_iota(jnp.int32, (_CH, 1), 0)
    nt = jnp.sum(jnp.where(tprob_ref[0] >= 0.5, jnp.float32(1.0), jnp.float32(0.0)))

    def argmin_rows(r0):
        mv = jnp.min(r0, axis=1, keepdims=True)
        mb = jnp.broadcast_to(mv, (_CH, _PAD))
        idxf = jnp.where(r0 == mb, lanef, _BIGIDX)
        iv = jnp.min(idxf, axis=1, keepdims=True)
        return mv, iv

    def dist_block(c):
        base = c * _CH
        t8 = txyz_ref[pl.ds(base, _CH), :]
        dot = jax.lax.dot_general(
            t8, pxyzT_ref[...],
            dimension_numbers=(((1,), (0,)), ((), ())),
            preferred_element_type=jnp.float32)
        tbase = base * 4
        a2v = jnp.zeros((_CH, 1), dtype=jnp.float32)
        a2s = []
        for k in range(_CH):
            off = tbase + 4 * k
            tx = traw_ref[off]
            ty = traw_ref[off + 1]
            tz = traw_ref[off + 2]
            tp = traw_ref[off + 3]
            a2k = tx * tx + ty * ty + tz * tz
            a2s.append(a2k)
            a2k = a2k + jnp.where(tp >= 0.5, jnp.float32(0.0), jnp.inf)
            a2v = jnp.where(row_i == k, a2k, a2v)
        d2 = (a2v + b2blk0) - 2.0 * dot
        dist = jnp.sqrt(jnp.maximum(d2, 0.0))
        return jnp.where(dist <= 1.0, dist, jnp.inf)

    def accum(k, base4, b_k, matched, nm, sdn, smse):
        off = base4 + 4 * k
        tx = traw_ref[off]
        ty = traw_ref[off + 1]
        tz = traw_ref[off + 2]
        tp = traw_ref[off + 3]
        idxi = jnp.maximum(b_k, jnp.float32(0.0)).astype(jnp.int32)
        pbase = idxi * 4
        pxi = praw_ref[pbase]
        pyi = praw_ref[pbase + 1]
        pzi = praw_ref[pbase + 2]
        ppi = praw_ref[pbase + 3]
        dx = tx - pxi
        dy = ty - pyi
        dz = tz - pzi
        dn = jnp.sqrt(dx * dx + dy * dy + dz * dz)
        mf = jnp.where(matched, jnp.float32(1.0), jnp.float32(0.0))
        dp = tp - ppi
        return nm + mf, sdn + mf * dn, smse + mf * dp * dp

    def chunk(c, carry):
        r0, takenv0, bprev, nm0, sdn0, smse0 = carry
        base4 = c * (_CH * 4)
        base4p = jnp.maximum(c - 1, 0) * (_CH * 4)
        nm, sdn, smse = nm0, sdn0, smse0
        for k in range(_CH):
            nm, sdn, smse = accum(k, base4p, bprev[k], bprev[k] >= 0.0,
                                  nm, sdn, smse)
        r0next = dist_block(jnp.minimum(c + 1, _NCHUNK - 1))
        mv8, iv8 = argmin_rows(r0 + takenv0)
        spec = [(mv8[k, 0], iv8[k, 0]) for k in range(_CH)]

        mats = [mv < jnp.inf for mv, _ in spec]
        bsp = [jnp.where(mats[k], spec[k][1], jnp.float32(-1.0))
               for k in range(_CH)]
        coll = jnp.bool_(False)
        for k in range(1, _CH):
            ck = jnp.bool_(False)
            for j in range(k):
                ck = jnp.logical_or(ck, spec[k][1] == bsp[j])
            coll = jnp.logical_or(coll, jnp.logical_and(mats[k], ck))

        def fast(takenv):
            masks = [lanef == bsp[k] for k in range(_CH)]
            while len(masks) > 1:
                nxt = [masks[i] | masks[i + 1] for i in range(0, len(masks) - 1, 2)]
                if len(masks) % 2:
                    nxt.append(masks[-1])
                masks = nxt
            takenv = jnp.where(masks[0], jnp.inf, takenv)
            return (takenv,) + tuple(bsp)

        def slow(takenv):
            bs = []
            for k in range(_CH):
                mv, ix = spec[k]
                ck = jnp.bool_(False)
                for j in range(k):
                    ck = jnp.logical_or(ck, ix == bs[j])
                ck = jnp.logical_and(mats[k], ck)

                def redo(tv, kk=k):
                    mv2, iv2 = argmin_rows(r0 + tv)
                    return mv2[kk, 0], iv2[kk, 0]

                def keep(tv, mv=mv, ix=ix):
                    return mv, ix

                mv, ix = jax.lax.cond(ck, redo, keep, takenv)
                matched = mv < jnp.inf
                b_k = jnp.where(matched, ix, jnp.float32(-1.0))
                bs.append(b_k)
                takenv = jnp.where(lanef == b_k, jnp.inf, takenv)
            return (takenv,) + tuple(bs)

        res = jax.lax.cond(coll, slow, fast, takenv0)
        return (r0next, res[0], list(res[1:]), nm, sdn, smse)

    z = jnp.float32(0.0)
    takenv_init = jnp.zeros((_CH, _PAD), dtype=jnp.float32)
    binit = [jnp.float32(-1.0)] * _CH
    _, _, blast, nm, sdn, smse = jax.lax.fori_loop(
        0, _NCHUNK, chunk, (dist_block(0), takenv_init, binit, z, z, z))
    base4l = (_NCHUNK - 1) * (_CH * 4)
    for k in range(_CH):
        nm, sdn, smse = accum(k, base4l, blast[k], blast[k] >= 0.0,
                              nm, sdn, smse)
    nu = nt - nm
    denom = jnp.maximum(nm, 1.0)
    extra = jnp.where(nm > 0.0, sdn / denom + smse / denom, 0.0)
    out_ref[0, 0] = 10.0 * nu + nu + extra


def kernel(pred_cloud, true_cloud):
    zpad = jnp.zeros((_PAD - _N, 4), dtype=pred_cloud.dtype)
    ppad = jnp.concatenate([pred_cloud, zpad], axis=0)
    pxyzT = jnp.transpose(ppad[:, :3])
    pprob = ppad[:, 3].reshape(1, _PAD)
    txyz = true_cloud[:, :3]
    tprob = jnp.concatenate(
        [true_cloud[:, 3], jnp.full((_PAD - _N,), -1.0, dtype=true_cloud.dtype)]
    ).reshape(1, 40, 128)
    out = pl.pallas_call(
        _loss_kernel,
        out_shape=jax.ShapeDtypeStruct((1, 1), jnp.float32),
        in_specs=[
            pl.BlockSpec(memory_space=pltpu.VMEM),
            pl.BlockSpec(memory_space=pltpu.VMEM),
            pl.BlockSpec(memory_space=pltpu.VMEM),
            pl.BlockSpec(memory_space=pltpu.SMEM),
            pl.BlockSpec(memory_space=pltpu.SMEM),
            pl.BlockSpec(memory_space=pltpu.VMEM),
        ],
        out_specs=pl.BlockSpec(memory_space=pltpu.SMEM),
    )(pxyzT, pprob, txyz, pred_cloud.reshape(-1), true_cloud.reshape(-1),
      tprob)
    return out.reshape(())
